# baseline (speedup 1.0000x reference)
import jax
import jax.numpy as jnp
from jax.experimental import pallas as pl
from jax.experimental.pallas import tpu as pltpu

T_CHUNK = 25


def _smcn_body(u_ref, x0_ref, wih_t_ref, whh_t_ref, b_ref, wf_t_ref, bf_ref,
               y_ref, x_ref, xs_ref):
    tc, bs, d_in = u_ref.shape
    d_out = y_ref.shape[-1]

    @pl.when(pl.program_id(0) == 0)
    def _():
        x_ref[...] = x0_ref[...]

    up = jnp.dot(u_ref[...].reshape(tc * bs, d_in), wih_t_ref[...],
                 preferred_element_type=jnp.float32) + b_ref[...]
    xs_ref[...] = up.reshape(tc, bs, d_in)

    whh_t = whh_t_ref[...]

    h = bs // 2
    x1 = x_ref[:h, :]
    x2 = x_ref[h:, :]
    for t in range(tc):
        x1 = jnp.tanh(xs_ref[t, :h, :] + jnp.dot(
            x1, whh_t, preferred_element_type=jnp.float32))
        x2 = jnp.tanh(xs_ref[t, h:, :] + jnp.dot(
            x2, whh_t, preferred_element_type=jnp.float32))
        xs_ref[t, :h, :] = x1
        xs_ref[t, h:, :] = x2
    x_ref[:h, :] = x1
    x_ref[h:, :] = x2

    ys = jnp.dot(xs_ref[...].reshape(tc * bs, d_in), wf_t_ref[...],
                 preferred_element_type=jnp.float32) + bf_ref[...]
    y_ref[...] = ys.reshape(tc, bs, 1, d_out)


def kernel(u, x0, W_ih, W_hh, b_ih, b_hh, W_f, b_f):
    T, BS, D_IN = u.shape
    D_OUT = W_f.shape[0]
    tc = T_CHUNK
    nblk = T // tc
    b = (b_ih + b_hh).reshape(1, D_IN)
    bf = b_f.reshape(1, D_OUT)
    y = pl.pallas_call(
        _smcn_body,
        grid=(nblk,),
        in_specs=[
            pl.BlockSpec((tc, BS, D_IN), lambda i: (i, 0, 0)),
            pl.BlockSpec((BS, D_IN), lambda i: (0, 0)),
            pl.BlockSpec((D_IN, D_IN), lambda i: (0, 0)),
            pl.BlockSpec((D_IN, D_IN), lambda i: (0, 0)),
            pl.BlockSpec((1, D_IN), lambda i: (0, 0)),
            pl.BlockSpec((D_IN, D_OUT), lambda i: (0, 0)),
            pl.BlockSpec((1, D_OUT), lambda i: (0, 0)),
        ],
        out_specs=pl.BlockSpec((tc, BS, 1, D_OUT), lambda i: (i, 0, 0, 0)),
        out_shape=jax.ShapeDtypeStruct((T, BS, 1, D_OUT), jnp.float32),
        scratch_shapes=[
            pltpu.VMEM((BS, D_IN), jnp.float32),
            pltpu.VMEM((tc, BS, D_IN), jnp.float32),
        ],
        compiler_params=pltpu.CompilerParams(
            dimension_semantics=("arbitrary",)),
    )(u, x0, W_ih.T, W_hh.T, b, W_f.T, bf)
    return y


# speedup vs baseline: 1.0656x; 1.0656x over previous
import jax
import jax.numpy as jnp
from jax.experimental import pallas as pl
from jax.experimental.pallas import tpu as pltpu

T_CHUNK = 25


def _smcn_body(u_ref, x0_ref, wih_t_ref, whh_t_ref, b_ref, wf_t_ref, bf_ref,
               y_ref, x_ref, xs_ref):
    tc, bs, d_in = u_ref.shape
    d_out = y_ref.shape[-1]

    @pl.when(pl.program_id(0) == 0)
    def _():
        x_ref[...] = x0_ref[...]

    up = jnp.dot(u_ref[...].reshape(tc * bs, d_in), wih_t_ref[...],
                 preferred_element_type=jnp.float32) + b_ref[...]
    xs_ref[...] = up.reshape(tc, bs, d_in)

    whh_t = whh_t_ref[...]

    h = bs // 2
    x1 = x_ref[:h, :]
    x2 = x_ref[h:, :]
    for t in range(tc):
        x1 = jnp.tanh(xs_ref[t, :h, :] + jnp.dot(
            x1, whh_t, preferred_element_type=jnp.float32))
        x2 = jnp.tanh(xs_ref[t, h:, :] + jnp.dot(
            x2, whh_t, preferred_element_type=jnp.float32))
        xs_ref[t, :h, :] = x1
        xs_ref[t, h:, :] = x2
    x_ref[:h, :] = x1
    x_ref[h:, :] = x2

    ys = jnp.dot(xs_ref[...].reshape(tc * bs, d_in), wf_t_ref[...],
                 preferred_element_type=jnp.float32) + bf_ref[...]
    y_ref[...] = ys.reshape(tc, 1, bs, d_out)


def kernel(u, x0, W_ih, W_hh, b_ih, b_hh, W_f, b_f):
    T, BS, D_IN = u.shape
    D_OUT = W_f.shape[0]
    tc = T_CHUNK
    nblk = T // tc
    b = (b_ih + b_hh).reshape(1, D_IN)
    bf = b_f.reshape(1, D_OUT)
    y = pl.pallas_call(
        _smcn_body,
        grid=(nblk,),
        in_specs=[
            pl.BlockSpec((tc, BS, D_IN), lambda i: (i, 0, 0)),
            pl.BlockSpec((BS, D_IN), lambda i: (0, 0)),
            pl.BlockSpec((D_IN, D_IN), lambda i: (0, 0)),
            pl.BlockSpec((D_IN, D_IN), lambda i: (0, 0)),
            pl.BlockSpec((1, D_IN), lambda i: (0, 0)),
            pl.BlockSpec((D_IN, D_OUT), lambda i: (0, 0)),
            pl.BlockSpec((1, D_OUT), lambda i: (0, 0)),
        ],
        out_specs=pl.BlockSpec((tc, 1, BS, D_OUT), lambda i: (i, 0, 0, 0)),
        out_shape=jax.ShapeDtypeStruct((T, 1, BS, D_OUT), jnp.float32),
        scratch_shapes=[
            pltpu.VMEM((BS, D_IN), jnp.float32),
            pltpu.VMEM((tc, BS, D_IN), jnp.float32),
        ],
        compiler_params=pltpu.CompilerParams(
            dimension_semantics=("arbitrary",)),
    )(u, x0, W_ih.T, W_hh.T, b, W_f.T, bf)
    return y.reshape(T, BS, 1, D_OUT)


# speedup vs baseline: 1.1107x; 1.0423x over previous
import jax
import jax.numpy as jnp
from jax.experimental import pallas as pl
from jax.experimental.pallas import tpu as pltpu

T_CHUNK = 25


def _smcn_body(u_ref, x0_ref, wih_ref, whh_ref, bih_ref, bhh_ref, wf_ref,
               bf_ref, y_ref, x_ref, xs_ref):
    tc, bs, d_in = u_ref.shape
    d_out = y_ref.shape[-1]

    @pl.when(pl.program_id(0) == 0)
    def _():
        x_ref[...] = x0_ref[...]

    wih_t = wih_ref[...].T
    whh_t = whh_ref[...].T
    wf_t = wf_ref[...].T
    badd = (bih_ref[...] + bhh_ref[...]).reshape(1, d_in)
    bf = bf_ref[...].reshape(1, d_out)

    up = jnp.dot(u_ref[...].reshape(tc * bs, d_in), wih_t,
                 preferred_element_type=jnp.float32) + badd
    xs_ref[...] = up.reshape(tc, bs, d_in)

    h = bs // 2
    x1 = x_ref[:h, :]
    x2 = x_ref[h:, :]
    for t in range(tc):
        x1 = jnp.tanh(xs_ref[t, :h, :] + jnp.dot(
            x1, whh_t, preferred_element_type=jnp.float32))
        x2 = jnp.tanh(xs_ref[t, h:, :] + jnp.dot(
            x2, whh_t, preferred_element_type=jnp.float32))
        xs_ref[t, :h, :] = x1
        xs_ref[t, h:, :] = x2
    x_ref[:h, :] = x1
    x_ref[h:, :] = x2

    ys = jnp.dot(xs_ref[...].reshape(tc * bs, d_in), wf_t,
                 preferred_element_type=jnp.float32) + bf
    y_ref[...] = ys.reshape(tc, bs, d_out)


def kernel(u, x0, W_ih, W_hh, b_ih, b_hh, W_f, b_f):
    T, BS, D_IN = u.shape
    D_OUT = W_f.shape[0]
    tc = T_CHUNK
    nblk = T // tc
    y = pl.pallas_call(
        _smcn_body,
        grid=(nblk,),
        in_specs=[
            pl.BlockSpec((tc, BS, D_IN), lambda i: (i, 0, 0)),
            pl.BlockSpec((BS, D_IN), lambda i: (0, 0)),
            pl.BlockSpec((D_IN, D_IN), lambda i: (0, 0)),
            pl.BlockSpec((D_IN, D_IN), lambda i: (0, 0)),
            pl.BlockSpec((D_IN,), lambda i: (0,)),
            pl.BlockSpec((D_IN,), lambda i: (0,)),
            pl.BlockSpec((D_OUT, D_IN), lambda i: (0, 0)),
            pl.BlockSpec((D_OUT,), lambda i: (0,)),
        ],
        out_specs=pl.BlockSpec((tc, BS, D_OUT), lambda i: (i, 0, 0)),
        out_shape=jax.ShapeDtypeStruct((T, BS, D_OUT), jnp.float32),
        scratch_shapes=[
            pltpu.VMEM((BS, D_IN), jnp.float32),
            pltpu.VMEM((tc, BS, D_IN), jnp.float32),
        ],
        compiler_params=pltpu.CompilerParams(
            dimension_semantics=("arbitrary",)),
    )(u, x0, W_ih, W_hh, b_ih, b_hh, W_f, b_f)
    return y.reshape(T, BS, 1, D_OUT)


# speedup vs baseline: 1.1167x; 1.0054x over previous
import jax
import jax.numpy as jnp
from jax.experimental import pallas as pl
from jax.experimental.pallas import tpu as pltpu

T_CHUNK = 40


def _smcn_body(u_ref, x0_ref, wih_ref, whh_ref, bih_ref, bhh_ref, wf_ref,
               bf_ref, y_ref, x_ref, xs_ref):
    tc, bs, d_in = u_ref.shape
    d_out = y_ref.shape[-1]

    @pl.when(pl.program_id(0) == 0)
    def _():
        x_ref[...] = x0_ref[...]

    wih_t = wih_ref[...].T
    whh_t = whh_ref[...].T
    wf_t = wf_ref[...].T
    badd = (bih_ref[...] + bhh_ref[...]).reshape(1, d_in)
    bf = bf_ref[...].reshape(1, d_out)

    up = jnp.dot(u_ref[...].reshape(tc * bs, d_in), wih_t,
                 preferred_element_type=jnp.float32) + badd
    xs_ref[...] = up.reshape(tc, bs, d_in)

    h = bs // 2
    x1 = x_ref[:h, :]
    x2 = x_ref[h:, :]
    for t in range(tc):
        x1 = jnp.tanh(xs_ref[t, :h, :] + jnp.dot(
            x1, whh_t, preferred_element_type=jnp.float32))
        x2 = jnp.tanh(xs_ref[t, h:, :] + jnp.dot(
            x2, whh_t, preferred_element_type=jnp.float32))
        xs_ref[t, :h, :] = x1
        xs_ref[t, h:, :] = x2
    x_ref[:h, :] = x1
    x_ref[h:, :] = x2

    ys = jnp.dot(xs_ref[...].reshape(tc * bs, d_in), wf_t,
                 preferred_element_type=jnp.float32) + bf
    y_ref[...] = ys.reshape(tc, bs, d_out)


def kernel(u, x0, W_ih, W_hh, b_ih, b_hh, W_f, b_f):
    T, BS, D_IN = u.shape
    D_OUT = W_f.shape[0]
    tc = T_CHUNK
    nblk = T // tc
    y = pl.pallas_call(
        _smcn_body,
        grid=(nblk,),
        in_specs=[
            pl.BlockSpec((tc, BS, D_IN), lambda i: (i, 0, 0)),
            pl.BlockSpec((BS, D_IN), lambda i: (0, 0)),
            pl.BlockSpec((D_IN, D_IN), lambda i: (0, 0)),
            pl.BlockSpec((D_IN, D_IN), lambda i: (0, 0)),
            pl.BlockSpec((D_IN,), lambda i: (0,)),
            pl.BlockSpec((D_IN,), lambda i: (0,)),
            pl.BlockSpec((D_OUT, D_IN), lambda i: (0, 0)),
            pl.BlockSpec((D_OUT,), lambda i: (0,)),
        ],
        out_specs=pl.BlockSpec((tc, BS, D_OUT), lambda i: (i, 0, 0)),
        out_shape=jax.ShapeDtypeStruct((T, BS, D_OUT), jnp.float32),
        scratch_shapes=[
            pltpu.VMEM((BS, D_IN), jnp.float32),
            pltpu.VMEM((tc, BS, D_IN), jnp.float32),
        ],
        compiler_params=pltpu.CompilerParams(
            dimension_semantics=("arbitrary",)),
    )(u, x0, W_ih, W_hh, b_ih, b_hh, W_f, b_f)
    return y.reshape(T, BS, 1, D_OUT)
